# Initial kernel scaffold; baseline (speedup 1.0000x reference)
import jax
import jax.numpy as jnp
from jax.experimental import pallas as pl


def kernel(x, edge_index, edge_weight, batch, W1, b1, W2, b2, Wl1, bl1, Wl2, bl2):
    raise NotImplementedError("write your pallas kernel here")


# speedup vs baseline: 1.0149x; 1.0149x over previous
import functools

import jax
import jax.numpy as jnp
from jax.experimental import pallas as pl
from jax.experimental.pallas import tpu as pltpu

N = 50000
E = 800000
B = 500
DIN = 90
H = 128
K = 70


def _head_body(hp_ref, wl1_ref, bl1_ref, wl2_ref, bl2_ref, o_ref, acc):
    j = pl.program_id(1)
    nk = pl.num_programs(1)

    @pl.when(j == 0)
    def _():
        acc[...] = jnp.zeros_like(acc)

    acc[...] += jnp.dot(hp_ref[...], wl1_ref[...],
                        preferred_element_type=jnp.float32)

    @pl.when(j == nk - 1)
    def _():
        t = acc[...] + bl1_ref[...]
        z = jnp.dot(t, wl2_ref[...], preferred_element_type=jnp.float32)
        o_ref[...] = jax.nn.sigmoid(z + bl2_ref[...])


def _dense_head(hp, Wl1, bl1, Wl2, bl2):
    MB = 128
    KB = 1280
    mpad = 512
    hp = jnp.pad(hp, ((0, mpad - B), (0, 0)))
    out = pl.pallas_call(
        _head_body,
        grid=(mpad // MB, (K * H) // KB),
        in_specs=[
            pl.BlockSpec((MB, KB), lambda i, j: (i, j)),
            pl.BlockSpec((KB, H), lambda i, j: (j, 0)),
            pl.BlockSpec((1, H), lambda i, j: (0, 0)),
            pl.BlockSpec((H, 1), lambda i, j: (0, 0)),
            pl.BlockSpec((1, 1), lambda i, j: (0, 0)),
        ],
        out_specs=pl.BlockSpec((MB, 1), lambda i, j: (i, 0)),
        out_shape=jax.ShapeDtypeStruct((mpad, 1), jnp.float32),
        scratch_shapes=[pltpu.VMEM((MB, H), jnp.float32)],
    )(hp, Wl1, bl1.reshape(1, H), Wl2, bl2.reshape(1, 1))
    return out[:B, 0]


def _gcn_conv(x, src, dst, ew, W, b):
    n = x.shape[0]
    loop = jnp.arange(n, dtype=src.dtype)
    s = jnp.concatenate([src, loop])
    d = jnp.concatenate([dst, loop])
    w = jnp.concatenate([ew, jnp.ones((n,), x.dtype)])
    deg = jnp.zeros((n,), x.dtype).at[d].add(w)
    dinv = jnp.where(deg > 0, jax.lax.rsqrt(deg), 0.0)
    norm = dinv[s] * w * dinv[d]
    xw = x @ W
    out = jnp.zeros((n, W.shape[1]), x.dtype).at[d].add(xw[s] * norm[:, None])
    return out + b


def _global_sort_pool(x, batch, pos, num_graphs, k):
    order = jnp.lexsort((-x[:, -1], batch))
    xs = x[order]
    dense = jnp.zeros((num_graphs, k, x.shape[1]), x.dtype).at[batch, pos].set(
        xs, mode="drop")
    return dense.reshape(num_graphs, k * x.shape[1])


def kernel(x, edge_index, edge_weight, batch, W1, b1, W2, b2, Wl1, bl1, Wl2, bl2):
    src, dst = edge_index[0], edge_index[1]
    counts = jnp.bincount(batch, length=B)
    csum = jnp.concatenate([jnp.zeros((1,), counts.dtype), jnp.cumsum(counts)])[:-1]
    pos = jnp.arange(N, dtype=batch.dtype) - csum[batch]
    h1 = _gcn_conv(x, src, dst, edge_weight, W1, b1)
    x_train = h1
    h = jax.nn.relu(h1)
    h = _gcn_conv(h, src, dst, edge_weight, W2, b2)
    hp = _global_sort_pool(h, batch, pos, B, K)
    out = _dense_head(hp, Wl1, bl1, Wl2, bl2)
    return (out, x_train)


# speedup vs baseline: 4.1123x; 4.0520x over previous
import functools

import jax
import jax.numpy as jnp
from jax import lax
from jax.experimental import pallas as pl
from jax.experimental.pallas import tpu as pltpu
from jax.experimental.pallas import tpu_sc as plsc

N = 50000
E = 800000
B = 500
DIN = 90
H = 128
K = 70

NP = 50176
EP = 802816
EPW = EP // 16
NBLK = EPW // 128
NPT = NP // 16
NZB = 392
CHW = 32


def _conv_sc_body(src_hbm, dst_hbm, w_hbm, xc_hbm, out_hbm,
                  sbuf, dbuf, wbuf, idxbuf, rows, zrow, obuf, acc, sem):
    c = lax.axis_index("c")
    s = lax.axis_index("s")

    def zr(i, carry):
        zrow[i, pl.ds(0, 16)] = jnp.zeros((16,), jnp.float32)
        zrow[i, pl.ds(16, 16)] = jnp.zeros((16,), jnp.float32)
        return carry
    lax.fori_loop(0, NZB, zr, 0)

    for p in range(2):
        ch = 2 * c + p
        base_off = ch * NP

        def zc(j, carry):
            pltpu.sync_copy(zrow, acc.at[pl.ds(s * NPT + j * NZB, NZB)])
            return carry
        lax.fori_loop(0, 8, zc, 0)
        plsc.subcore_barrier()

        def blk(i, carry):
            ebase = s * EPW + i * 128
            pltpu.sync_copy(src_hbm.at[pl.ds(ebase, 128)], sbuf)
            pltpu.sync_copy(dst_hbm.at[pl.ds(ebase, 128)], dbuf)
            pltpu.sync_copy(w_hbm.at[pl.ds(ebase, 128)], wbuf)

            def addo(q, cy):
                idxbuf[pl.ds(q * 16, 16)] = sbuf[pl.ds(q * 16, 16)] + base_off
                return cy
            lax.fori_loop(0, 8, addo, 0)
            pltpu.async_copy(xc_hbm.at[idxbuf], rows, sem).wait()

            def sc(q, cy):
                wvec = wbuf[pl.ds(q * 16, 16)]
                for l in range(16):
                    e = q * 16 + l
                    wv = wvec[l]
                    rows[e, pl.ds(0, 16)] = rows[e, pl.ds(0, 16)] * wv
                    rows[e, pl.ds(16, 16)] = rows[e, pl.ds(16, 16)] * wv
                return cy
            lax.fori_loop(0, 8, sc, 0)
            pltpu.sync_copy(rows, acc.at[dbuf], add=True)
            return carry
        lax.fori_loop(0, NBLK, blk, 0)
        plsc.subcore_barrier()

        def wo(j, carry):
            r0 = s * NPT + j * NZB
            pltpu.sync_copy(acc.at[pl.ds(r0, NZB)], obuf)
            pltpu.sync_copy(obuf, out_hbm.at[pl.ds(base_off + r0, NZB)])
            return carry
        lax.fori_loop(0, 8, wo, 0)
        plsc.subcore_barrier()


_conv_sc = pl.kernel(
    _conv_sc_body,
    out_type=jax.ShapeDtypeStruct((4 * NP, CHW), jnp.float32),
    mesh=plsc.VectorSubcoreMesh(core_axis_name="c", subcore_axis_name="s"),
    scratch_types=[
        pltpu.VMEM((128,), jnp.int32),
        pltpu.VMEM((128,), jnp.int32),
        pltpu.VMEM((128,), jnp.float32),
        pltpu.VMEM((128,), jnp.int32),
        pltpu.VMEM((128, CHW), jnp.float32),
        pltpu.VMEM((NZB, CHW), jnp.float32),
        pltpu.VMEM((NZB, CHW), jnp.float32),
        pltpu.VMEM_SHARED((NP, CHW), jnp.float32),
        pltpu.SemaphoreType.DMA,
    ],
    compiler_params=pltpu.CompilerParams(use_tc_tiling_on_sc=False),
)


def _to_chunks(xp):
    return jnp.transpose(xp.reshape(NP, 4, CHW), (1, 0, 2)).reshape(4 * NP, CHW)


def _from_chunks(agg):
    return jnp.transpose(agg.reshape(4, NP, CHW), (1, 0, 2)).reshape(NP, H)


def _head_body(hp_ref, wl1_ref, bl1_ref, wl2_ref, bl2_ref, o_ref, acc):
    j = pl.program_id(1)
    nk = pl.num_programs(1)

    @pl.when(j == 0)
    def _():
        acc[...] = jnp.zeros_like(acc)

    acc[...] += jnp.dot(hp_ref[...], wl1_ref[...],
                        preferred_element_type=jnp.float32)

    @pl.when(j == nk - 1)
    def _():
        t = acc[...] + bl1_ref[...]
        z = jnp.dot(t, wl2_ref[...], preferred_element_type=jnp.float32)
        o_ref[...] = jax.nn.sigmoid(z + bl2_ref[...])


def _dense_head(hp, Wl1, bl1, Wl2, bl2):
    MB = 128
    KB = 1280
    mpad = 512
    hp = jnp.pad(hp, ((0, mpad - B), (0, 0)))
    out = pl.pallas_call(
        _head_body,
        grid=(mpad // MB, (K * H) // KB),
        in_specs=[
            pl.BlockSpec((MB, KB), lambda i, j: (i, j)),
            pl.BlockSpec((KB, H), lambda i, j: (j, 0)),
            pl.BlockSpec((1, H), lambda i, j: (0, 0)),
            pl.BlockSpec((H, 1), lambda i, j: (0, 0)),
            pl.BlockSpec((1, 1), lambda i, j: (0, 0)),
        ],
        out_specs=pl.BlockSpec((MB, 1), lambda i, j: (i, 0)),
        out_shape=jax.ShapeDtypeStruct((mpad, 1), jnp.float32),
        scratch_shapes=[pltpu.VMEM((MB, H), jnp.float32)],
    )(hp, Wl1, bl1.reshape(1, H), Wl2, bl2.reshape(1, 1))
    return out[:B, 0]


def _global_sort_pool(x, batch, pos, num_graphs, k):
    order = jnp.lexsort((-x[:, -1], batch))
    xs = x[order]
    dense = jnp.zeros((num_graphs, k, x.shape[1]), x.dtype).at[batch, pos].set(
        xs, mode="drop")
    return dense.reshape(num_graphs, k * x.shape[1])


def kernel(x, edge_index, edge_weight, batch, W1, b1, W2, b2, Wl1, bl1, Wl2, bl2):
    src, dst = edge_index[0], edge_index[1]
    src_p = jnp.pad(src, (0, EP - E))
    dst_p = jnp.pad(dst, (0, EP - E))
    w_p = jnp.pad(edge_weight, (0, EP - E))

    deg = jnp.ones((N,), jnp.float32).at[dst].add(edge_weight)
    dinv = lax.rsqrt(deg)

    x1 = (x @ W1) * dinv[:, None]
    x1p = jnp.pad(x1, ((0, NP - N), (0, 0)))
    agg1 = _conv_sc(src_p, dst_p, w_p, _to_chunks(x1p))
    h1 = dinv[:, None] * (_from_chunks(agg1)[:N] + x1) + b1
    x_train = h1

    h = jax.nn.relu(h1)
    x2 = (h @ W2) * dinv[:, None]
    x2p = jnp.pad(x2, ((0, NP - N), (0, 0)))
    agg2 = _conv_sc(src_p, dst_p, w_p, _to_chunks(x2p))
    h2 = dinv[:, None] * (_from_chunks(agg2)[:N] + x2) + b2

    counts = jnp.bincount(batch, length=B)
    csum = jnp.concatenate([jnp.zeros((1,), counts.dtype), jnp.cumsum(counts)])[:-1]
    pos = jnp.arange(N, dtype=batch.dtype) - csum[batch]
    hp = _global_sort_pool(h2, batch, pos, B, K)
    out = _dense_head(hp, Wl1, bl1, Wl2, bl2)
    return (out, x_train)
